# baseline (speedup 1.0000x reference)
import jax
import jax.numpy as jnp
from jax.experimental import pallas as pl
from jax.experimental.pallas import tpu as pltpu

_C = 10000
_NBUF = 2


def _make_body(M, K, O):
    nch = M // _C

    def body(x_hbm, w_ref, o_hbm, x_buf, y_buf, load_sem, store_sem):
        def load(i, slot):
            return pltpu.make_async_copy(
                x_hbm.at[pl.ds(i * _C, _C), :],
                x_buf.at[slot],
                load_sem.at[slot],
            )

        def store(i, slot):
            return pltpu.make_async_copy(
                y_buf.at[slot],
                o_hbm.at[pl.ds(i * _C, _C), :],
                store_sem.at[slot],
            )

        for b in range(min(_NBUF, nch)):
            load(b, b).start()

        for i in range(nch):
            slot = i % _NBUF
            load(i, slot).wait()
            if i >= _NBUF:
                store(i - _NBUF, slot).wait()
            y_buf[slot] = jax.lax.dot_general(
                x_buf[slot],
                w_ref[...],
                dimension_numbers=(((1,), (1,)), ((), ())),
                preferred_element_type=jnp.float32,
            )
            store(i, slot).start()
            if i + _NBUF < nch:
                load(i + _NBUF, slot).start()

        for i in range(max(nch - _NBUF, 0), nch):
            store(i, i % _NBUF).wait()

    return body


def kernel(x, W):
    M, K = x.shape
    O = W.shape[0]
    return pl.pallas_call(
        _make_body(M, K, O),
        in_specs=[
            pl.BlockSpec(memory_space=pl.ANY),
            pl.BlockSpec((O, K), lambda: (0, 0)),
        ],
        out_specs=pl.BlockSpec(memory_space=pl.ANY),
        out_shape=jax.ShapeDtypeStruct((M, O), jnp.float32),
        scratch_shapes=[
            pltpu.VMEM((_NBUF, _C, K), jnp.float32),
            pltpu.VMEM((_NBUF, _C, O), jnp.float32),
            pltpu.SemaphoreType.DMA((_NBUF,)),
            pltpu.SemaphoreType.DMA((_NBUF,)),
        ],
    )(x, W)


# speedup vs baseline: 1.1001x; 1.1001x over previous
import jax
import jax.numpy as jnp
from jax.experimental import pallas as pl
from jax.experimental.pallas import tpu as pltpu

_BM = 14912


def _linear_kernel(x_ref, w_ref, o_ref):
    o_ref[...] = jax.lax.dot_general(
        x_ref[...],
        w_ref[...],
        dimension_numbers=(((1,), (1,)), ((), ())),
        preferred_element_type=jnp.float32,
    )


def kernel(x, W):
    M, K = x.shape
    O = W.shape[0]
    return pl.pallas_call(
        _linear_kernel,
        grid=(pl.cdiv(M, _BM),),
        in_specs=[
            pl.BlockSpec((_BM, K), lambda i: (i, 0)),
            pl.BlockSpec((O, K), lambda i: (0, 0)),
        ],
        out_specs=pl.BlockSpec((_BM, O), lambda i: (i, 0)),
        out_shape=jax.ShapeDtypeStruct((M, O), jnp.float32),
        compiler_params=pltpu.CompilerParams(
            dimension_semantics=("parallel",),
        ),
    )(x, W)
